# baseline (device time: 40573 ns/iter reference)
import jax
import jax.numpy as jnp
from jax import lax
from jax.experimental import pallas as pl
from jax.experimental.pallas import tpu as pltpu

N_DEV = 4
H_GLOBAL = 512
W = 128
N_NORM = H_GLOBAL * W
EPS = 1e-5


def kernel(x, Wp):
    b, h_per, w, c = x.shape
    c_out = Wp.shape[1]

    xt = jnp.transpose(x, (0, 1, 3, 2))

    def body(xt_ref, wp_ref, out_ref, stats_ref, send_sems, recv_sems):
        my = lax.axis_index("i")

        barrier_sem = pltpu.get_barrier_semaphore()
        for d in range(1, N_DEV):
            pl.semaphore_signal(
                barrier_sem, inc=1,
                device_id=((my + d) % N_DEV,),
                device_id_type=pl.DeviceIdType.MESH,
            )

        xtv = xt_ref[...]

        s = jnp.sum(jnp.sum(xtv, axis=3), axis=1)
        sq = jnp.sum(jnp.sum(xtv * xtv, axis=3), axis=1)
        stats_ref[0, :, :] = jnp.concatenate(
            [s, sq, jnp.zeros((8 - 2 * b, c), jnp.float32)], axis=0
        )

        pl.semaphore_wait(barrier_sem, N_DEV - 1)

        rdmas = []
        for d in range(1, N_DEV):
            rdma = pltpu.make_async_remote_copy(
                src_ref=stats_ref.at[0],
                dst_ref=stats_ref.at[N_DEV - d],
                send_sem=send_sems.at[d - 1],
                recv_sem=recv_sems.at[N_DEV - d],
                device_id=((my + d) % N_DEV,),
                device_id_type=pl.DeviceIdType.MESH,
            )
            rdma.start()
            rdmas.append(rdma)

        xv = jnp.swapaxes(xtv, 2, 3)
        x3 = xv.reshape(b, h_per * w, c)

        for rdma in rdmas:
            rdma.wait()

        total = jnp.sum(stats_ref[...], axis=0)

        inv_n = jnp.float32(1.0 / N_NORM)
        mean = total[0:b, :] * inv_n
        ex2 = total[b:2 * b, :] * inv_n
        rstd = lax.rsqrt(ex2 - mean * mean + EPS)
        hh = (x3 - mean[:, None, :]) * rstd[:, None, :]
        a = hh * (1.0 / (1.0 + jnp.exp(-hh)))
        ob = jnp.dot(
            a.reshape(b * h_per * w, c), wp_ref[...],
            preferred_element_type=jnp.float32,
        )
        out_ref[...] = ob.reshape(b, h_per, w, c_out)

    return pl.pallas_call(
        body,
        out_shape=jax.ShapeDtypeStruct((b, h_per, w, c_out), jnp.float32),
        in_specs=[
            pl.BlockSpec(memory_space=pltpu.VMEM),
            pl.BlockSpec(memory_space=pltpu.VMEM),
        ],
        out_specs=pl.BlockSpec(memory_space=pltpu.VMEM),
        scratch_shapes=[
            pltpu.VMEM((N_DEV, 8, 64), jnp.float32),
            pltpu.SemaphoreType.DMA((N_DEV - 1,)),
            pltpu.SemaphoreType.DMA((N_DEV,)),
        ],
        compiler_params=pltpu.CompilerParams(collective_id=0),
    )(xt, Wp)


# device time: 39109 ns/iter; 1.0374x vs baseline; 1.0374x over previous
import jax
import jax.numpy as jnp
from jax import lax
from jax.experimental import pallas as pl
from jax.experimental.pallas import tpu as pltpu

N_DEV = 4
H_GLOBAL = 512
W = 128
N_NORM = H_GLOBAL * W
EPS = 1e-5


def kernel(x, Wp):
    b, h_per, w, c = x.shape
    c_out = Wp.shape[1]

    xt = jnp.transpose(x, (0, 1, 3, 2))

    def body(xt_ref, wp_ref, out_ref, stats_ref, obuf_ref, send_sems,
             recv_sems, out_sems):
        my = lax.axis_index("i")

        barrier_sem = pltpu.get_barrier_semaphore()
        for d in range(1, N_DEV):
            pl.semaphore_signal(
                barrier_sem, inc=1,
                device_id=((my + d) % N_DEV,),
                device_id_type=pl.DeviceIdType.MESH,
            )

        xtv = xt_ref[...]

        s = jnp.sum(jnp.sum(xtv, axis=3), axis=1)
        sq = jnp.sum(jnp.sum(xtv * xtv, axis=3), axis=1)
        stats_ref[0, :, :] = jnp.concatenate(
            [s, sq, jnp.zeros((8 - 2 * b, c), jnp.float32)], axis=0
        )

        pl.semaphore_wait(barrier_sem, N_DEV - 1)

        rdmas = []
        for d in range(1, N_DEV):
            rdma = pltpu.make_async_remote_copy(
                src_ref=stats_ref.at[0],
                dst_ref=stats_ref.at[N_DEV - d],
                send_sem=send_sems.at[d - 1],
                recv_sem=recv_sems.at[N_DEV - d],
                device_id=((my + d) % N_DEV,),
                device_id_type=pl.DeviceIdType.MESH,
            )
            rdma.start()
            rdmas.append(rdma)

        xv = jnp.swapaxes(xtv, 2, 3)
        x3 = xv.reshape(b, h_per * w, c)

        for rdma in rdmas:
            rdma.wait()

        total = jnp.sum(stats_ref[...], axis=0)

        inv_n = jnp.float32(1.0 / N_NORM)
        mean = total[0:b, :] * inv_n
        ex2 = total[b:2 * b, :] * inv_n
        rstd = lax.rsqrt(ex2 - mean * mean + EPS)

        hc = h_per // 2
        wpv = wp_ref[...]
        for k in range(4):
            bb, hk = divmod(k, 2)
            xk = xv[bb, hk * hc:(hk + 1) * hc].reshape(hc * w, c)
            hh = (xk - mean[bb:bb + 1, :]) * rstd[bb:bb + 1, :]
            a = hh * (1.0 / (1.0 + jnp.exp(-hh)))
            ob = jnp.dot(a, wpv, preferred_element_type=jnp.float32)
            obuf_ref[k % 2] = ob.reshape(hc, w, c_out)
            copy = pltpu.make_async_copy(
                obuf_ref.at[k % 2],
                out_ref.at[bb, pl.ds(hk * hc, hc)],
                out_sems.at[k % 2],
            )
            copy.start()
            if k >= 1:
                prev_bb, prev_hk = divmod(k - 1, 2)
                pltpu.make_async_copy(
                    obuf_ref.at[(k - 1) % 2],
                    out_ref.at[prev_bb, pl.ds(prev_hk * hc, hc)],
                    out_sems.at[(k - 1) % 2],
                ).wait()
        pltpu.make_async_copy(
            obuf_ref.at[1], out_ref.at[1, pl.ds(hc, hc)], out_sems.at[1]
        ).wait()

    return pl.pallas_call(
        body,
        out_shape=jax.ShapeDtypeStruct((b, h_per, w, c_out), jnp.float32),
        in_specs=[
            pl.BlockSpec(memory_space=pltpu.VMEM),
            pl.BlockSpec(memory_space=pltpu.VMEM),
        ],
        out_specs=pl.BlockSpec(memory_space=pl.ANY),
        scratch_shapes=[
            pltpu.VMEM((N_DEV, 8, 64), jnp.float32),
            pltpu.VMEM((2, h_per // 2, w, c_out), jnp.float32),
            pltpu.SemaphoreType.DMA((N_DEV - 1,)),
            pltpu.SemaphoreType.DMA((N_DEV,)),
            pltpu.SemaphoreType.DMA((2,)),
        ],
        compiler_params=pltpu.CompilerParams(collective_id=0),
    )(xt, Wp)


# device time: 36481 ns/iter; 1.1122x vs baseline; 1.0720x over previous
import jax
import jax.numpy as jnp
from jax import lax
from jax.experimental import pallas as pl
from jax.experimental.pallas import tpu as pltpu

N_DEV = 4
H_GLOBAL = 512
W = 128
N_NORM = H_GLOBAL * W
EPS = 1e-5


def kernel(x, Wp):
    b, h_per, w, c = x.shape
    c_out = Wp.shape[1]

    xt = jnp.transpose(x, (0, 1, 3, 2))

    def body(xt_ref, wp_ref, out_ref, stats_ref, send_sems, recv_sems):
        my = lax.axis_index("i")

        barrier_sem = pltpu.get_barrier_semaphore()
        for d in range(1, N_DEV):
            pl.semaphore_signal(
                barrier_sem, inc=1,
                device_id=((my + d) % N_DEV,),
                device_id_type=pl.DeviceIdType.MESH,
            )

        xtv = xt_ref[...]

        s = jnp.sum(jnp.sum(xtv, axis=3), axis=1)
        sq = jnp.sum(jnp.sum(xtv * xtv, axis=3), axis=1)
        stats_ref[0, :, :] = jnp.concatenate(
            [s, sq, jnp.zeros((8 - 2 * b, c), jnp.float32)], axis=0
        )

        pl.semaphore_wait(barrier_sem, N_DEV - 1)

        rdmas = []
        for d in range(1, N_DEV):
            rdma = pltpu.make_async_remote_copy(
                src_ref=stats_ref.at[0],
                dst_ref=stats_ref.at[N_DEV - d],
                send_sem=send_sems.at[d - 1],
                recv_sem=recv_sems.at[N_DEV - d],
                device_id=((my + d) % N_DEV,),
                device_id_type=pl.DeviceIdType.MESH,
            )
            rdma.start()
            rdmas.append(rdma)

        for rdma in rdmas:
            rdma.wait()

        total = jnp.sum(stats_ref[...], axis=0)

        inv_n = jnp.float32(1.0 / N_NORM)
        mean = total[0:b, :] * inv_n
        ex2 = total[b:2 * b, :] * inv_n
        rstd = lax.rsqrt(ex2 - mean * mean + EPS)

        hh = (xtv - mean[:, None, :, None]) * rstd[:, None, :, None]
        a4 = hh * (1.0 / (1.0 + jnp.exp(-hh)))

        n_chunks = 4
        hc = h_per // n_chunks
        wpv = wp_ref[...]
        for k in range(b * n_chunks):
            bb, hk = divmod(k, n_chunks)
            ak = a4[bb, hk * hc:(hk + 1) * hc]
            akt = jnp.swapaxes(ak, 1, 2)
            ob = jnp.dot(
                akt.reshape(hc * w, c), wpv,
                preferred_element_type=jnp.float32,
            )
            out_ref[bb, hk * hc:(hk + 1) * hc] = ob.reshape(hc, w, c_out)

    return pl.pallas_call(
        body,
        out_shape=jax.ShapeDtypeStruct((b, h_per, w, c_out), jnp.float32),
        in_specs=[
            pl.BlockSpec(memory_space=pltpu.VMEM),
            pl.BlockSpec(memory_space=pltpu.VMEM),
        ],
        out_specs=pl.BlockSpec(memory_space=pltpu.VMEM),
        scratch_shapes=[
            pltpu.VMEM((N_DEV, 8, 64), jnp.float32),
            pltpu.SemaphoreType.DMA((N_DEV - 1,)),
            pltpu.SemaphoreType.DMA((N_DEV,)),
        ],
        compiler_params=pltpu.CompilerParams(collective_id=0),
    )(xt, Wp)
